# baseline (speedup 1.0000x reference)
import jax
import jax.numpy as jnp
from jax.experimental import pallas as pl
from jax.experimental.pallas import tpu as pltpu

N_BARS = 256
B_STATIC = 64
EMB_D = 1024
N_SEM = 8


def _body(shift_ref, emb_ref, out_ref, gath_ref, gath2_ref, sems):
    shift = shift_ref[0]
    emb = emb_ref[...]
    rolled = pltpu.roll(emb, -shift, 0)
    pos = jax.lax.broadcasted_iota(jnp.int32, (N_BARS, EMB_D), 0) + shift
    oob = (pos >= N_BARS) | (pos < -N_BARS)
    gath = jnp.where(oob, jnp.nan, rolled)
    gath_ref[...] = gath
    gath2_ref[...] = gath
    srcs = (gath_ref, gath2_ref)
    copies = [
        pltpu.make_async_copy(srcs[j % 2], out_ref.at[j], sems.at[j % N_SEM])
        for j in range(B_STATIC)
    ]
    for c in copies:
        c.start()
    for c in copies:
        c.wait()


def kernel(num_bars, batch_size, embedding):
    shift = (num_bars - N_BARS) + (batch_size - B_STATIC)
    shift_arr = jnp.asarray(shift, jnp.int32).reshape(1)

    out = pl.pallas_call(
        _body,
        in_specs=[
            pl.BlockSpec(memory_space=pltpu.SMEM),
            pl.BlockSpec(memory_space=pltpu.VMEM),
        ],
        out_specs=pl.BlockSpec(memory_space=pl.ANY),
        out_shape=jax.ShapeDtypeStruct((B_STATIC, N_BARS, EMB_D), jnp.float32),
        scratch_shapes=[
            pltpu.VMEM((N_BARS, EMB_D), jnp.float32),
            pltpu.VMEM((N_BARS, EMB_D), jnp.float32),
            pltpu.SemaphoreType.DMA((N_SEM,)),
        ],
    )(shift_arr, embedding)
    return out


# speedup vs baseline: 1.0388x; 1.0388x over previous
import jax
import jax.numpy as jnp
from jax.experimental import pallas as pl
from jax.experimental.pallas import tpu as pltpu

N_BARS = 256
B_STATIC = 64
EMB_D = 1024
N_SEM = 8


def _body(shift_ref, emb_ref, out_ref, gath_ref, sems):
    shift = shift_ref[0]

    @pl.when(shift == 0)
    def _fast():
        for j in range(B_STATIC):
            pltpu.make_async_copy(emb_ref, out_ref.at[j],
                                  sems.at[j % N_SEM]).start()

    @pl.when(shift != 0)
    def _general():
        emb = emb_ref[...]
        rolled = pltpu.roll(emb, -shift, 0)
        pos = jax.lax.broadcasted_iota(jnp.int32, (N_BARS, EMB_D), 0) + shift
        oob = (pos >= N_BARS) | (pos < -N_BARS)
        gath_ref[...] = jnp.where(oob, jnp.nan, rolled)
        for j in range(B_STATIC):
            pltpu.make_async_copy(gath_ref, out_ref.at[j],
                                  sems.at[j % N_SEM]).start()

    for j in range(B_STATIC):
        pltpu.make_async_copy(gath_ref, out_ref.at[j],
                              sems.at[j % N_SEM]).wait()


def kernel(num_bars, batch_size, embedding):
    shift = (num_bars - N_BARS) + (batch_size - B_STATIC)
    shift_arr = jnp.asarray(shift, jnp.int32).reshape(1)

    out = pl.pallas_call(
        _body,
        in_specs=[
            pl.BlockSpec(memory_space=pltpu.SMEM),
            pl.BlockSpec(memory_space=pltpu.VMEM),
        ],
        out_specs=pl.BlockSpec(memory_space=pl.ANY),
        out_shape=jax.ShapeDtypeStruct((B_STATIC, N_BARS, EMB_D), jnp.float32),
        scratch_shapes=[
            pltpu.VMEM((N_BARS, EMB_D), jnp.float32),
            pltpu.SemaphoreType.DMA((N_SEM,)),
        ],
    )(shift_arr, embedding)
    return out
